# baseline (speedup 1.0000x reference)
import functools

import jax
import jax.numpy as jnp
from jax import lax
from jax.experimental import pallas as pl
from jax.experimental.pallas import tpu as pltpu
from jax.experimental.pallas import tpu_sc as plsc

NC = 2
NS = 16
NW = NC * NS
LANES = 16
ROW_UNROLL = 4


@functools.lru_cache(maxsize=None)
def _build(B, T, D):
    tpw = T // NW
    mesh = plsc.VectorSubcoreMesh(core_axis_name="c", subcore_axis_name="s")

    @functools.partial(
        pl.kernel,
        out_type=jax.ShapeDtypeStruct((B * T, D), jnp.float32),
        mesh=mesh,
        scratch_types=[
            pltpu.VMEM((B, tpw), jnp.int32),
            pltpu.VMEM((B, tpw, D), jnp.float32),
            pltpu.VMEM((tpw, D), jnp.float32),
            pltpu.SemaphoreType.DMA((B,)),
            pltpu.SemaphoreType.DMA((B,)),
            pltpu.SemaphoreType.DMA,
        ],
    )
    def sc_kernel(idx_hbm, tok_hbm, pos_hbm, out_hbm, idx_v, rows_v, pos_v,
                  gsem, osem, psem):
        c = lax.axis_index("c")
        s = lax.axis_index("s")
        wid = s * NC + c
        tbase = wid * tpw

        for b in range(B):
            pltpu.sync_copy(idx_hbm.at[b, pl.ds(tbase, tpw)], idx_v.at[b])

        gathers = [
            pltpu.async_copy(tok_hbm.at[idx_v.at[b]], rows_v.at[b], gsem.at[b])
            for b in range(B)
        ]
        pos_cp = pltpu.async_copy(pos_hbm.at[pl.ds(tbase, tpw)], pos_v, psem)

        stores = []
        for b in range(B):
            gathers[b].wait()
            if b == 0:
                pos_cp.wait()

            def add_rows(i, carry, b=b):
                for u in range(ROW_UNROLL):
                    r = i * ROW_UNROLL + u
                    for ch in range(D // LANES):
                        sl = pl.ds(ch * LANES, LANES)
                        rows_v[b, r, sl] = rows_v[b, r, sl] + pos_v[r, sl]
                return carry

            lax.fori_loop(0, tpw // ROW_UNROLL, add_rows, 0)
            stores.append(
                pltpu.async_copy(
                    rows_v.at[b], out_hbm.at[pl.ds(b * T + tbase, tpw)],
                    osem.at[b],
                )
            )
        for st in stores:
            st.wait()

    return sc_kernel


def kernel(idx, token_table, pos_table):
    B, T = idx.shape
    V, D = token_table.shape
    tpw = T // NW
    assert T % NW == 0 and tpw % 8 == 0 and tpw <= 128
    assert tpw % ROW_UNROLL == 0 and D % LANES == 0

    out = _build(B, T, D)(idx.astype(jnp.int32), token_table, pos_table)
    return out.reshape(B, T, D)


# speedup vs baseline: 1.0372x; 1.0372x over previous
import functools

import jax
import jax.numpy as jnp
from jax import lax
from jax.experimental import pallas as pl
from jax.experimental.pallas import tpu as pltpu
from jax.experimental.pallas import tpu_sc as plsc

NC = 2
NS = 16
NW = NC * NS
LANES = 16
ROW_UNROLL = 4


@functools.lru_cache(maxsize=None)
def _build(B, T, D):
    tpw = T // NW
    mesh = plsc.VectorSubcoreMesh(core_axis_name="c", subcore_axis_name="s")

    @functools.partial(
        pl.kernel,
        out_type=jax.ShapeDtypeStruct((B * T, D), jnp.float32),
        mesh=mesh,
        scratch_types=[
            pltpu.VMEM((B, tpw), jnp.int32),
            pltpu.VMEM((B, tpw, D), jnp.float32),
            pltpu.VMEM((tpw, D), jnp.float32),
            pltpu.SemaphoreType.DMA((B,)),
            pltpu.SemaphoreType.DMA((B,)),
            pltpu.SemaphoreType.DMA,
        ],
    )
    def sc_kernel(idx_hbm, tok_hbm, pos_hbm, out_hbm, idx_v, rows_v, pos_v,
                  gsem, osem, psem):
        c = lax.axis_index("c")
        s = lax.axis_index("s")
        wid = s * NC + c
        tbase = wid * tpw

        pltpu.sync_copy(idx_hbm.at[wid], idx_v)

        gathers = [
            pltpu.async_copy(tok_hbm.at[idx_v.at[b]], rows_v.at[b], gsem.at[b])
            for b in range(B)
        ]
        pos_cp = pltpu.async_copy(pos_hbm.at[pl.ds(tbase, tpw)], pos_v, psem)

        stores = []
        for b in range(B):
            gathers[b].wait()
            if b == 0:
                pos_cp.wait()

            def add_rows(i, carry, b=b):
                for u in range(ROW_UNROLL):
                    r = i * ROW_UNROLL + u
                    for ch in range(D // LANES):
                        sl = pl.ds(ch * LANES, LANES)
                        rows_v[b, r, sl] = rows_v[b, r, sl] + pos_v[r, sl]
                return carry

            lax.fori_loop(0, tpw // ROW_UNROLL, add_rows, 0)
            stores.append(
                pltpu.async_copy(
                    rows_v.at[b], out_hbm.at[pl.ds(b * T + tbase, tpw)],
                    osem.at[b],
                )
            )
        for st in stores:
            st.wait()

    return sc_kernel


def kernel(idx, token_table, pos_table):
    B, T = idx.shape
    V, D = token_table.shape
    tpw = T // NW
    assert T % NW == 0 and tpw % 8 == 0 and tpw <= 128
    assert tpw % ROW_UNROLL == 0 and D % LANES == 0

    idx_r = idx.astype(jnp.int32).reshape(B, NW, tpw).transpose(1, 0, 2)
    out = _build(B, T, D)(idx_r, token_table, pos_table)
    return out.reshape(B, T, D)


# speedup vs baseline: 1.0917x; 1.0526x over previous
import functools

import jax
import jax.numpy as jnp
from jax import lax
from jax.experimental import pallas as pl
from jax.experimental.pallas import tpu as pltpu
from jax.experimental.pallas import tpu_sc as plsc

NC = 2
NS = 16
NW = NC * NS
LANES = 16
RB = 16


@functools.lru_cache(maxsize=None)
def _build(B, T, D):
    tpw = T // NW
    mesh = plsc.VectorSubcoreMesh(core_axis_name="c", subcore_axis_name="s")

    @functools.partial(
        pl.kernel,
        out_type=jax.ShapeDtypeStruct((B * T, D), jnp.float32),
        mesh=mesh,
        scratch_types=[
            pltpu.VMEM((B, tpw), jnp.int32),
            pltpu.VMEM((B, tpw, D), jnp.float32),
            pltpu.VMEM((tpw, D), jnp.float32),
            pltpu.SemaphoreType.DMA((B,)),
            pltpu.SemaphoreType.DMA((B,)),
        ],
    )
    def sc_kernel(idx_hbm, tok_hbm, pos_hbm, out_hbm, idx_v, rows_v, pos_v,
                  gsem, osem):
        c = lax.axis_index("c")
        s = lax.axis_index("s")
        wid = s * NC + c
        tbase = wid * tpw

        pltpu.sync_copy(idx_hbm.at[wid], idx_v)

        gathers = [
            pltpu.async_copy(tok_hbm.at[idx_v.at[b]], rows_v.at[b], gsem.at[b])
            for b in range(B)
        ]
        pltpu.sync_copy(pos_hbm.at[pl.ds(tbase, tpw)], pos_v)
        for g in gathers:
            g.wait()

        stores = []
        for rb in range(tpw // RB):

            def add_row(r, carry):
                prow = [pos_v[r, pl.ds(ch * LANES, LANES)]
                        for ch in range(D // LANES)]
                for b in range(B):
                    for ch in range(D // LANES):
                        plsc.addupdate(
                            rows_v.at[b, r, pl.ds(ch * LANES, LANES)],
                            prow[ch],
                        )
                return carry

            lax.fori_loop(rb * RB, (rb + 1) * RB, add_row, 0)
            for b in range(B):
                stores.append(
                    pltpu.async_copy(
                        rows_v.at[b, pl.ds(rb * RB, RB)],
                        out_hbm.at[pl.ds(b * T + tbase + rb * RB, RB)],
                        osem.at[b],
                    )
                )
        for st in stores:
            st.wait()

    return sc_kernel


def kernel(idx, token_table, pos_table):
    B, T = idx.shape
    V, D = token_table.shape
    tpw = T // NW
    assert T % NW == 0 and tpw % RB == 0 and tpw <= 128 and D % LANES == 0

    idx_r = idx.astype(jnp.int32).reshape(B, NW, tpw).transpose(1, 0, 2)
    out = _build(B, T, D)(idx_r, token_table, pos_table)
    return out.reshape(B, T, D)


# speedup vs baseline: 1.0937x; 1.0018x over previous
import functools

import jax
import jax.numpy as jnp
from jax import lax
from jax.experimental import pallas as pl
from jax.experimental.pallas import tpu as pltpu
from jax.experimental.pallas import tpu_sc as plsc

NC = 2
NS = 16
NW = NC * NS
LANES = 16
RB = 16


@functools.lru_cache(maxsize=None)
def _build(B, T, D):
    tpw = T // NW
    mesh = plsc.VectorSubcoreMesh(core_axis_name="c", subcore_axis_name="s")

    @functools.partial(
        pl.kernel,
        out_type=jax.ShapeDtypeStruct((B * T, D), jnp.float32),
        mesh=mesh,
        scratch_types=[
            pltpu.VMEM((B, tpw), jnp.int32),
            pltpu.VMEM((B, tpw, D), jnp.float32),
            pltpu.VMEM((tpw, D), jnp.float32),
            pltpu.SemaphoreType.DMA((B,)),
            pltpu.SemaphoreType.DMA((B,)),
            pltpu.SemaphoreType.DMA((B,)),
        ],
    )
    def sc_kernel(idx_hbm, tok_hbm, pos_hbm, out_hbm, idx_v, rows_v, pos_v,
                  gsem, osem, isem):
        c = lax.axis_index("c")
        s = lax.axis_index("s")
        wid = s * NC + c
        tbase = wid * tpw

        idx_cps = [
            pltpu.async_copy(idx_hbm.at[b, pl.ds(tbase, tpw)], idx_v.at[b],
                             isem.at[b])
            for b in range(B)
        ]
        gathers = []
        for b in range(B):
            idx_cps[b].wait()
            gathers.append(
                pltpu.async_copy(tok_hbm.at[idx_v.at[b]], rows_v.at[b],
                                 gsem.at[b])
            )
        pltpu.sync_copy(pos_hbm.at[pl.ds(tbase, tpw)], pos_v)
        for g in gathers:
            g.wait()

        stores = []
        for rb in range(tpw // RB):

            def add_row(r, carry):
                prow = [pos_v[r, pl.ds(ch * LANES, LANES)]
                        for ch in range(D // LANES)]
                for b in range(B):
                    for ch in range(D // LANES):
                        plsc.addupdate(
                            rows_v.at[b, r, pl.ds(ch * LANES, LANES)],
                            prow[ch],
                        )
                return carry

            lax.fori_loop(rb * RB, (rb + 1) * RB, add_row, 0)
            for b in range(B):
                stores.append(
                    pltpu.async_copy(
                        rows_v.at[b, pl.ds(rb * RB, RB)],
                        out_hbm.at[pl.ds(b * T + tbase + rb * RB, RB)],
                        osem.at[b],
                    )
                )
        for st in stores:
            st.wait()

    return sc_kernel


def kernel(idx, token_table, pos_table):
    B, T = idx.shape
    V, D = token_table.shape
    tpw = T // NW
    assert T % NW == 0 and tpw % RB == 0 and tpw <= 128 and D % LANES == 0

    out = _build(B, T, D)(idx.astype(jnp.int32), token_table, pos_table)
    return out.reshape(B, T, D)
